# baseline (speedup 1.0000x reference)
import jax
import jax.numpy as jnp
from jax.experimental import pallas as pl
from jax.experimental.pallas import tpu as pltpu

_B, _C, _H, _W = 16, 96, 24, 24
_P = _H * _W
_KP = 500 // _B
_KPAD = 32


def _body(optf_ref, sarf_ref, muW_ref, mub_ref, lvW_ref, lvb_ref,
          f1W_ref, f1b_ref, f2W_ref, f2b_ref, r1W_ref, r1b_ref,
          r2W_ref, r2b_ref, eps_ref,
          supd_ref, sc_ref, recon_ref, totkl_ref,
          omu_ref, olv_ref, pm_ref, plv_ref, od_ref, ob_ref, nrow_ref,
          idx_ref, acc_ref):
    b = pl.program_id(0)
    opt = optf_ref[0]
    sar = sarf_ref[0]
    muW = muW_ref[...]
    mub = mub_ref[...]
    lvW = lvW_ref[...]
    lvb = lvb_ref[...]

    omu = jnp.dot(opt, muW, preferred_element_type=jnp.float32) + mub
    olv = jnp.dot(opt, lvW, preferred_element_type=jnp.float32) + lvb
    smu = jnp.dot(sar, muW, preferred_element_type=jnp.float32) + mub
    slv = jnp.clip(jnp.dot(sar, lvW, preferred_element_type=jnp.float32) + lvb,
                   -10.0, 10.0)
    omu_ref[...] = omu
    olv_ref[...] = olv

    sum_o = jnp.sum(olv, axis=1, keepdims=True)
    sum_s = jnp.sum(slv, axis=1, keepdims=True)
    vcol = 0.5 * jnp.log((jnp.exp(sum_s) + 1e-6) / (jnp.exp(sum_o) + 1e-6))

    h1 = jnp.maximum(
        jax.lax.dot_general(vcol, f1W_ref[...], (((0,), (0,)), ((), ())),
                            preferred_element_type=jnp.float32)
        + f1b_ref[...], 0.0)
    raw = jnp.dot(h1, f2W_ref[...], preferred_element_type=jnp.float32) + f2b_ref[...]
    mx = jnp.max(raw, axis=1, keepdims=True)
    ex = jnp.exp(raw - mx)
    scores = ex / jnp.sum(ex, axis=1, keepdims=True)
    sc_ref[0] = scores

    pmx = jnp.max(omu, axis=1, keepdims=True)
    pex = jnp.exp(omu - pmx)
    psum = jnp.sum(pex, axis=1, keepdims=True)
    p = pex / psum
    qmx = jnp.max(smu, axis=1, keepdims=True)
    logq = (smu - qmx) - jnp.log(jnp.sum(jnp.exp(smu - qmx), axis=1, keepdims=True))
    align_part = jnp.sum(p * (jnp.log(p) - logq))

    supd_ref[0] = sar

    iot = jax.lax.broadcasted_iota(jnp.int32, (1, _P), 1)
    s = scores
    for t in range(_KP):
        mv = jnp.max(s, axis=1, keepdims=True)
        cand = jnp.where(s >= mv, iot, jnp.int32(1_000_000))
        it = jnp.min(cand)
        idx_ref[t] = it
        s = jnp.where(iot == it, -jnp.inf, s)

    zrow = jnp.zeros((1, _C), jnp.float32)
    pm_ref[pl.ds(_KP, 1), :] = zrow
    plv_ref[pl.ds(_KP, 1), :] = zrow
    od_ref[pl.ds(_KP, 1), :] = zrow
    ob_ref[pl.ds(_KP, 1), :] = zrow
    for t in range(_KP):
        i = idx_ref[t]
        pm_ref[pl.ds(t, 1), :] = omu_ref[pl.ds(i, 1), :]
        plv_ref[pl.ds(t, 1), :] = olv_ref[pl.ds(i, 1), :]
        od_ref[pl.ds(t, 1), :] = sarf_ref[0, pl.ds(i, 1), :]
        ob_ref[pl.ds(t, 1), :] = optf_ref[0, pl.ds(i, 1), :]

    pmv = pm_ref[...]
    plvv = plv_ref[...]
    oldv = od_ref[...]
    optv = ob_ref[...]

    z = pmv + jnp.exp(0.5 * plvv) * eps_ref[0]
    h = jnp.maximum(
        jnp.dot(z, r1W_ref[...], preferred_element_type=jnp.float32) + r1b_ref[...],
        0.0)
    rec = jnp.dot(h, r2W_ref[...], preferred_element_type=jnp.float32) + r2b_ref[...]
    newr = 0.5 * rec + 0.5 * oldv
    nrow_ref[...] = newr

    rmask = (jax.lax.broadcasted_iota(jnp.int32, (_KPAD, 1), 0) < _KP
             ).astype(jnp.float32)
    d = newr - optv
    recon_part = jnp.sum(d * d * rmask)
    kl_part = jnp.sum((1.0 + plvv - pmv * pmv - jnp.exp(plvv)) * rmask)

    @pl.when(b == 0)
    def _init():
        acc_ref[0] = 0.0
        acc_ref[1] = 0.0
        acc_ref[2] = 0.0

    acc_ref[0] = acc_ref[0] + recon_part
    acc_ref[1] = acc_ref[1] + kl_part
    acc_ref[2] = acc_ref[2] + align_part

    for t in range(_KP):
        i = idx_ref[t]
        supd_ref[0, pl.ds(i, 1), :] = nrow_ref[pl.ds(t, 1), :]

    @pl.when(b == _B - 1)
    def _fin():
        recon_ref[0, 0] = acc_ref[0] * (1.0 / (_B * _KP * _C))
        totkl_ref[0, 0] = (acc_ref[1] * (-0.5 / _B) + acc_ref[2] * (1.0 / _B))


def kernel(opt_token, sar_token, mu_W, mu_b, lv_W, lv_b, fc1_W, fc1_b,
           fc2_W, fc2_b, rec1_W, rec1_b, rec2_W, rec2_b):
    optf = opt_token.reshape(_B, _C, _P).transpose(0, 2, 1)
    sarf = sar_token.reshape(_B, _C, _P).transpose(0, 2, 1)
    eps = jax.random.normal(jax.random.key(42), (_B * _KP, _C), jnp.float32)
    eps_p = jnp.zeros((_B, _KPAD, _C), jnp.float32
                      ).at[:, :_KP].set(eps.reshape(_B, _KP, _C))

    full = lambda *shape: pl.BlockSpec(shape, lambda b: (0,) * len(shape))
    in_specs = [
        pl.BlockSpec((1, _P, _C), lambda b: (b, 0, 0)),
        pl.BlockSpec((1, _P, _C), lambda b: (b, 0, 0)),
        full(_C, _C), full(1, _C),
        full(_C, _C), full(1, _C),
        full(_P, 128), full(1, 128),
        full(128, _P), full(1, _P),
        full(_C, 128), full(1, 128),
        full(128, _C), full(1, _C),
        pl.BlockSpec((1, _KPAD, _C), lambda b: (b, 0, 0)),
    ]
    out_specs = [
        pl.BlockSpec((1, _P, _C), lambda b: (b, 0, 0)),
        pl.BlockSpec((1, 1, _P), lambda b: (b, 0, 0)),
        pl.BlockSpec((1, 1), lambda b: (0, 0), memory_space=pltpu.SMEM),
        pl.BlockSpec((1, 1), lambda b: (0, 0), memory_space=pltpu.SMEM),
    ]
    out_shapes = [
        jax.ShapeDtypeStruct((_B, _P, _C), jnp.float32),
        jax.ShapeDtypeStruct((_B, 1, _P), jnp.float32),
        jax.ShapeDtypeStruct((1, 1), jnp.float32),
        jax.ShapeDtypeStruct((1, 1), jnp.float32),
    ]
    scratch = [
        pltpu.VMEM((_P, _C), jnp.float32),
        pltpu.VMEM((_P, _C), jnp.float32),
        pltpu.VMEM((_KPAD, _C), jnp.float32),
        pltpu.VMEM((_KPAD, _C), jnp.float32),
        pltpu.VMEM((_KPAD, _C), jnp.float32),
        pltpu.VMEM((_KPAD, _C), jnp.float32),
        pltpu.VMEM((_KPAD, _C), jnp.float32),
        pltpu.SMEM((_KPAD,), jnp.int32),
        pltpu.SMEM((3,), jnp.float32),
    ]
    sar_upd, scores, recon, totkl = pl.pallas_call(
        _body,
        grid=(_B,),
        in_specs=in_specs,
        out_specs=out_specs,
        out_shape=out_shapes,
        scratch_shapes=scratch,
        compiler_params=pltpu.CompilerParams(
            dimension_semantics=("arbitrary",)),
    )(optf, sarf, mu_W, mu_b.reshape(1, _C), lv_W, lv_b.reshape(1, _C),
      fc1_W, fc1_b.reshape(1, 128), fc2_W, fc2_b.reshape(1, _P),
      rec1_W, rec1_b.reshape(1, 128), rec2_W, rec2_b.reshape(1, _C), eps_p)

    unc_map = scores.reshape(_B, 1, _H, _W)
    return (optf, sar_upd, recon[0, 0], totkl[0, 0], unc_map)


# speedup vs baseline: 1.0179x; 1.0179x over previous
import jax
import jax.numpy as jnp
from jax.experimental import pallas as pl
from jax.experimental.pallas import tpu as pltpu

_B, _C, _H, _W = 16, 96, 24, 24
_P = _H * _W
_KP = 500 // _B
_KPAD = 32

_EXACT = jax.lax.Precision.HIGHEST
_TDIMS = (((0,), (0,)), ((), ()))


def _body(optf_ref, sarf_ref, muW_ref, mub_ref, lvW_ref, lvb_ref,
          f1W_ref, f1b_ref, f2W_ref, f2b_ref, r1W_ref, r1b_ref,
          r2W_ref, r2b_ref, eps_ref,
          supd_ref, sc_ref, recon_ref, totkl_ref, acc_ref):
    b = pl.program_id(0)
    opt = optf_ref[0]
    sar = sarf_ref[0]
    muW = muW_ref[...]
    mub = mub_ref[...]
    lvW = lvW_ref[...]
    lvb = lvb_ref[...]

    omu = jnp.dot(opt, muW, preferred_element_type=jnp.float32) + mub
    olv = jnp.dot(opt, lvW, preferred_element_type=jnp.float32) + lvb
    smu = jnp.dot(sar, muW, preferred_element_type=jnp.float32) + mub
    slv = jnp.clip(jnp.dot(sar, lvW, preferred_element_type=jnp.float32) + lvb,
                   -10.0, 10.0)

    sum_o = jnp.sum(olv, axis=1, keepdims=True)
    sum_s = jnp.sum(slv, axis=1, keepdims=True)
    vcol = 0.5 * jnp.log((jnp.exp(sum_s) + 1e-6) / (jnp.exp(sum_o) + 1e-6))

    h1 = jnp.maximum(
        jax.lax.dot_general(vcol, f1W_ref[...], _TDIMS,
                            preferred_element_type=jnp.float32)
        + f1b_ref[...], 0.0)
    raw = jnp.dot(h1, f2W_ref[...], preferred_element_type=jnp.float32) + f2b_ref[...]
    mx = jnp.max(raw, axis=1, keepdims=True)
    ex = jnp.exp(raw - mx)
    scores = ex / jnp.sum(ex, axis=1, keepdims=True)
    sc_ref[0] = scores

    pmx = jnp.max(omu, axis=1, keepdims=True)
    pex = jnp.exp(omu - pmx)
    p = pex / jnp.sum(pex, axis=1, keepdims=True)
    qmx = jnp.max(smu, axis=1, keepdims=True)
    logq = (smu - qmx) - jnp.log(jnp.sum(jnp.exp(smu - qmx), axis=1, keepdims=True))
    align_part = jnp.sum(p * (jnp.log(p) - logq))

    iot = jax.lax.broadcasted_iota(jnp.int32, (1, _P), 1)
    rowid = jax.lax.broadcasted_iota(jnp.int32, (_KPAD, 1), 0)
    idxv = jnp.full((_KPAD, 1), -1, jnp.int32)
    s = scores
    for t in range(_KP):
        mv = jnp.max(s, axis=1, keepdims=True)
        cand = jnp.where(s >= mv, iot, jnp.int32(1_000_000))
        itv = jnp.min(cand, axis=1, keepdims=True)
        idxv = jnp.where(rowid == t, itv, idxv)
        s = jnp.where(iot == itv, -jnp.inf, s)

    sel = (jax.lax.broadcasted_iota(jnp.int32, (_KPAD, _P), 1) == idxv
           ).astype(jnp.float32)

    pmu = jnp.dot(sel, omu, precision=_EXACT, preferred_element_type=jnp.float32)
    plv = jnp.dot(sel, olv, precision=_EXACT, preferred_element_type=jnp.float32)
    old = jnp.dot(sel, sar, precision=_EXACT, preferred_element_type=jnp.float32)
    osel = jnp.dot(sel, opt, precision=_EXACT, preferred_element_type=jnp.float32)

    z = pmu + jnp.exp(0.5 * plv) * eps_ref[0]
    h = jnp.maximum(
        jnp.dot(z, r1W_ref[...], preferred_element_type=jnp.float32) + r1b_ref[...],
        0.0)
    rec = jnp.dot(h, r2W_ref[...], preferred_element_type=jnp.float32) + r2b_ref[...]
    newr = 0.5 * rec + 0.5 * old

    delta = jax.lax.dot_general(sel, newr - old, _TDIMS, precision=_EXACT,
                                preferred_element_type=jnp.float32)
    supd_ref[0] = sar + delta

    rmask = (rowid < _KP).astype(jnp.float32)
    d = newr - osel
    recon_part = jnp.sum(d * d * rmask)
    kl_part = jnp.sum((1.0 + plv - pmu * pmu - jnp.exp(plv)) * rmask)

    @pl.when(b == 0)
    def _init():
        acc_ref[0] = 0.0
        acc_ref[1] = 0.0
        acc_ref[2] = 0.0

    acc_ref[0] = acc_ref[0] + recon_part
    acc_ref[1] = acc_ref[1] + kl_part
    acc_ref[2] = acc_ref[2] + align_part

    @pl.when(b == _B - 1)
    def _fin():
        recon_ref[0, 0] = acc_ref[0] * (1.0 / (_B * _KP * _C))
        totkl_ref[0, 0] = (acc_ref[1] * (-0.5 / _B) + acc_ref[2] * (1.0 / _B))


def kernel(opt_token, sar_token, mu_W, mu_b, lv_W, lv_b, fc1_W, fc1_b,
           fc2_W, fc2_b, rec1_W, rec1_b, rec2_W, rec2_b):
    optf = opt_token.reshape(_B, _C, _P).transpose(0, 2, 1)
    sarf = sar_token.reshape(_B, _C, _P).transpose(0, 2, 1)
    eps = jax.random.normal(jax.random.key(42), (_B * _KP, _C), jnp.float32)
    eps_p = jnp.zeros((_B, _KPAD, _C), jnp.float32
                      ).at[:, :_KP].set(eps.reshape(_B, _KP, _C))

    full = lambda *shape: pl.BlockSpec(shape, lambda b: (0,) * len(shape))
    in_specs = [
        pl.BlockSpec((1, _P, _C), lambda b: (b, 0, 0)),
        pl.BlockSpec((1, _P, _C), lambda b: (b, 0, 0)),
        full(_C, _C), full(1, _C),
        full(_C, _C), full(1, _C),
        full(_P, 128), full(1, 128),
        full(128, _P), full(1, _P),
        full(_C, 128), full(1, 128),
        full(128, _C), full(1, _C),
        pl.BlockSpec((1, _KPAD, _C), lambda b: (b, 0, 0)),
    ]
    out_specs = [
        pl.BlockSpec((1, _P, _C), lambda b: (b, 0, 0)),
        pl.BlockSpec((1, 1, _P), lambda b: (b, 0, 0)),
        pl.BlockSpec((1, 1), lambda b: (0, 0), memory_space=pltpu.SMEM),
        pl.BlockSpec((1, 1), lambda b: (0, 0), memory_space=pltpu.SMEM),
    ]
    out_shapes = [
        jax.ShapeDtypeStruct((_B, _P, _C), jnp.float32),
        jax.ShapeDtypeStruct((_B, 1, _P), jnp.float32),
        jax.ShapeDtypeStruct((1, 1), jnp.float32),
        jax.ShapeDtypeStruct((1, 1), jnp.float32),
    ]
    scratch = [
        pltpu.SMEM((3,), jnp.float32),
    ]
    sar_upd, scores, recon, totkl = pl.pallas_call(
        _body,
        grid=(_B,),
        in_specs=in_specs,
        out_specs=out_specs,
        out_shape=out_shapes,
        scratch_shapes=scratch,
        compiler_params=pltpu.CompilerParams(
            dimension_semantics=("arbitrary",)),
    )(optf, sarf, mu_W, mu_b.reshape(1, _C), lv_W, lv_b.reshape(1, _C),
      fc1_W, fc1_b.reshape(1, 128), fc2_W, fc2_b.reshape(1, _P),
      rec1_W, rec1_b.reshape(1, 128), rec2_W, rec2_b.reshape(1, _C), eps_p)

    unc_map = scores.reshape(_B, 1, _H, _W)
    return (optf, sar_upd, recon[0, 0], totkl[0, 0], unc_map)


# speedup vs baseline: 2.6993x; 2.6518x over previous
import jax
import jax.numpy as jnp
from jax.experimental import pallas as pl
from jax.experimental.pallas import tpu as pltpu

_B, _C, _H, _W = 16, 96, 24, 24
_P = _H * _W
_KP = 500 // _B
_KPAD = 32

_EXACT = jax.lax.Precision.HIGHEST
_TDIMS = (((0,), (0,)), ((), ()))


def _body(optf_ref, sarf_ref, muW_ref, mub_ref, lvW_ref, lvb_ref,
          f1W_ref, f1b_ref, f2W_ref, f2b_ref, r1W_ref, r1b_ref,
          r2W_ref, r2b_ref, eps_ref, eye_ref,
          supd_ref, sc_ref, recon_ref, totkl_ref, acc_ref):
    b = pl.program_id(0)
    opt = optf_ref[0]
    sar = sarf_ref[0]
    muW = muW_ref[...]
    mub = mub_ref[...]
    lvW = lvW_ref[...]
    lvb = lvb_ref[...]

    omu = jnp.dot(opt, muW, preferred_element_type=jnp.float32) + mub
    olv = jnp.dot(opt, lvW, preferred_element_type=jnp.float32) + lvb
    smu = jnp.dot(sar, muW, preferred_element_type=jnp.float32) + mub
    slv = jnp.clip(jnp.dot(sar, lvW, preferred_element_type=jnp.float32) + lvb,
                   -10.0, 10.0)

    sum_o = jnp.sum(olv, axis=1, keepdims=True)
    sum_s = jnp.sum(slv, axis=1, keepdims=True)
    vcol = 0.5 * jnp.log((jnp.exp(sum_s) + 1e-6) / (jnp.exp(sum_o) + 1e-6))

    h1 = jnp.maximum(
        jax.lax.dot_general(vcol, f1W_ref[...], _TDIMS,
                            preferred_element_type=jnp.float32)
        + f1b_ref[...], 0.0)
    raw = jnp.dot(h1, f2W_ref[...], preferred_element_type=jnp.float32) + f2b_ref[...]
    mx = jnp.max(raw, axis=1, keepdims=True)
    ex = jnp.exp(raw - mx)
    scores = ex / jnp.sum(ex, axis=1, keepdims=True)
    sc_ref[0] = scores

    pmx = jnp.max(omu, axis=1, keepdims=True)
    pex = jnp.exp(omu - pmx)
    p = pex / jnp.sum(pex, axis=1, keepdims=True)
    qmx = jnp.max(smu, axis=1, keepdims=True)
    logq = (smu - qmx) - jnp.log(jnp.sum(jnp.exp(smu - qmx), axis=1, keepdims=True))
    align_part = jnp.sum(p * (jnp.log(p) - logq))

    scol = jax.lax.dot_general(eye_ref[...], scores, (((1,), (1,)), ((), ())),
                               precision=_EXACT,
                               preferred_element_type=jnp.float32)
    qi = jax.lax.broadcasted_iota(jnp.int32, (_P, 1), 0)
    pi = jax.lax.broadcasted_iota(jnp.int32, (1, _P), 1)
    beats = (scol > scores) | ((scol == scores) & (qi < pi))
    rank = jnp.sum(jnp.where(beats, 1.0, 0.0), axis=0, keepdims=True)

    rowid = jax.lax.broadcasted_iota(jnp.int32, (_KPAD, 1), 0)
    rmask = (rowid < _KP).astype(jnp.float32)
    sel = jnp.where(rank == rowid.astype(jnp.float32), 1.0, 0.0) * rmask

    pmu = jnp.dot(sel, omu, precision=_EXACT, preferred_element_type=jnp.float32)
    plv = jnp.dot(sel, olv, precision=_EXACT, preferred_element_type=jnp.float32)
    old = jnp.dot(sel, sar, precision=_EXACT, preferred_element_type=jnp.float32)
    osel = jnp.dot(sel, opt, precision=_EXACT, preferred_element_type=jnp.float32)

    z = pmu + jnp.exp(0.5 * plv) * eps_ref[0]
    h = jnp.maximum(
        jnp.dot(z, r1W_ref[...], preferred_element_type=jnp.float32) + r1b_ref[...],
        0.0)
    rec = jnp.dot(h, r2W_ref[...], preferred_element_type=jnp.float32) + r2b_ref[...]
    newr = 0.5 * rec + 0.5 * old

    delta = jax.lax.dot_general(sel, newr - old, _TDIMS, precision=_EXACT,
                                preferred_element_type=jnp.float32)
    supd_ref[0] = sar + delta

    d = newr - osel
    recon_part = jnp.sum(d * d * rmask)
    kl_part = jnp.sum((1.0 + plv - pmu * pmu - jnp.exp(plv)) * rmask)

    @pl.when(b == 0)
    def _init():
        acc_ref[0] = 0.0
        acc_ref[1] = 0.0
        acc_ref[2] = 0.0

    acc_ref[0] = acc_ref[0] + recon_part
    acc_ref[1] = acc_ref[1] + kl_part
    acc_ref[2] = acc_ref[2] + align_part

    @pl.when(b == _B - 1)
    def _fin():
        recon_ref[0, 0] = acc_ref[0] * (1.0 / (_B * _KP * _C))
        totkl_ref[0, 0] = (acc_ref[1] * (-0.5 / _B) + acc_ref[2] * (1.0 / _B))


def kernel(opt_token, sar_token, mu_W, mu_b, lv_W, lv_b, fc1_W, fc1_b,
           fc2_W, fc2_b, rec1_W, rec1_b, rec2_W, rec2_b):
    optf = opt_token.reshape(_B, _C, _P).transpose(0, 2, 1)
    sarf = sar_token.reshape(_B, _C, _P).transpose(0, 2, 1)
    eps = jax.random.normal(jax.random.key(42), (_B * _KP, _C), jnp.float32)
    eps_p = jnp.zeros((_B, _KPAD, _C), jnp.float32
                      ).at[:, :_KP].set(eps.reshape(_B, _KP, _C))

    full = lambda *shape: pl.BlockSpec(shape, lambda b: (0,) * len(shape))
    in_specs = [
        pl.BlockSpec((1, _P, _C), lambda b: (b, 0, 0)),
        pl.BlockSpec((1, _P, _C), lambda b: (b, 0, 0)),
        full(_C, _C), full(1, _C),
        full(_C, _C), full(1, _C),
        full(_P, 128), full(1, 128),
        full(128, _P), full(1, _P),
        full(_C, 128), full(1, 128),
        full(128, _C), full(1, _C),
        pl.BlockSpec((1, _KPAD, _C), lambda b: (b, 0, 0)),
        full(_P, _P),
    ]
    out_specs = [
        pl.BlockSpec((1, _P, _C), lambda b: (b, 0, 0)),
        pl.BlockSpec((1, 1, _P), lambda b: (b, 0, 0)),
        pl.BlockSpec((1, 1), lambda b: (0, 0), memory_space=pltpu.SMEM),
        pl.BlockSpec((1, 1), lambda b: (0, 0), memory_space=pltpu.SMEM),
    ]
    out_shapes = [
        jax.ShapeDtypeStruct((_B, _P, _C), jnp.float32),
        jax.ShapeDtypeStruct((_B, 1, _P), jnp.float32),
        jax.ShapeDtypeStruct((1, 1), jnp.float32),
        jax.ShapeDtypeStruct((1, 1), jnp.float32),
    ]
    scratch = [
        pltpu.SMEM((3,), jnp.float32),
    ]
    sar_upd, scores, recon, totkl = pl.pallas_call(
        _body,
        grid=(_B,),
        in_specs=in_specs,
        out_specs=out_specs,
        out_shape=out_shapes,
        scratch_shapes=scratch,
        compiler_params=pltpu.CompilerParams(
            dimension_semantics=("arbitrary",)),
    )(optf, sarf, mu_W, mu_b.reshape(1, _C), lv_W, lv_b.reshape(1, _C),
      fc1_W, fc1_b.reshape(1, 128), fc2_W, fc2_b.reshape(1, _P),
      rec1_W, rec1_b.reshape(1, 128), rec2_W, rec2_b.reshape(1, _C), eps_p,
      jnp.eye(_P, dtype=jnp.float32))

    unc_map = scores.reshape(_B, 1, _H, _W)
    return (optf, sar_upd, recon[0, 0], totkl[0, 0], unc_map)


# speedup vs baseline: 3.4242x; 1.2685x over previous
import jax
import jax.numpy as jnp
import numpy as np
from jax.experimental import pallas as pl
from jax.experimental.pallas import tpu as pltpu

_B, _C, _H, _W = 16, 96, 24, 24
_P = _H * _W
_KP = 500 // _B
_KPAD = 32
_NI = 2
_STEPS = _B // _NI

_EXACT = jax.lax.Precision.HIGHEST
_TDIMS = (((0,), (0,)), ((), ()))


def _body(optf_ref, sarf_ref, muW_ref, mub_ref, lvW_ref, lvb_ref,
          f1W_ref, f1b_ref, f2W_ref, f2b_ref, r1W_ref, r1b_ref,
          r2W_ref, r2b_ref, eps_ref,
          supd_ref, sc_ref, recon_ref, totkl_ref, cat_ref, acc_ref):
    b = pl.program_id(0)
    opt2 = optf_ref[...]
    sar2 = sarf_ref[...]
    opt = opt2.reshape(_NI * _P, _C)
    sar = sar2.reshape(_NI * _P, _C)
    muW = muW_ref[...]
    mub = mub_ref[...]
    lvW = lvW_ref[...]
    lvb = lvb_ref[...]

    omu = jnp.dot(opt, muW, preferred_element_type=jnp.float32) + mub
    olv = jnp.dot(opt, lvW, preferred_element_type=jnp.float32) + lvb
    smu = jnp.dot(sar, muW, preferred_element_type=jnp.float32) + mub
    slv = jnp.clip(jnp.dot(sar, lvW, preferred_element_type=jnp.float32) + lvb,
                   -10.0, 10.0)

    sum_o = jnp.sum(olv, axis=1, keepdims=True)
    sum_s = jnp.sum(slv, axis=1, keepdims=True)
    vcol = 0.5 * jnp.log((jnp.exp(sum_s) + 1e-6) / (jnp.exp(sum_o) + 1e-6))

    h1s = [
        jnp.maximum(
            jax.lax.dot_general(vcol[i * _P:(i + 1) * _P], f1W_ref[...],
                                _TDIMS, preferred_element_type=jnp.float32)
            + f1b_ref[...], 0.0)
        for i in range(_NI)
    ]
    raw = (jnp.dot(jnp.concatenate(h1s, axis=0), f2W_ref[...],
                   preferred_element_type=jnp.float32)
           + f2b_ref[...])
    mx = jnp.max(raw, axis=1, keepdims=True)
    ex = jnp.exp(raw - mx)
    scores = ex / jnp.sum(ex, axis=1, keepdims=True)
    sc_ref[...] = scores.reshape(_NI, 1, _P)

    pmx = jnp.max(omu, axis=1, keepdims=True)
    pex = jnp.exp(omu - pmx)
    p = pex / jnp.sum(pex, axis=1, keepdims=True)
    qmx = jnp.max(smu, axis=1, keepdims=True)
    logq = (smu - qmx) - jnp.log(jnp.sum(jnp.exp(smu - qmx), axis=1, keepdims=True))
    align_part = jnp.sum(p * (jnp.log(p) - logq))

    scol2 = jnp.transpose(scores)

    cat_ref[:, 0:_C] = omu
    cat_ref[:, 128:128 + _C] = olv
    cat_ref[:, 256:256 + _C] = sar
    cat_ref[:, 384:384 + _C] = opt

    qi = jax.lax.broadcasted_iota(jnp.int32, (_P, 1), 0)
    pi = jax.lax.broadcasted_iota(jnp.int32, (1, _P), 1)
    rowid = jax.lax.broadcasted_iota(jnp.int32, (_KPAD, 1), 0)
    rmask = (rowid < _KP).astype(jnp.float32)

    sels = []
    gath = []
    for i in range(_NI):
        srow = scores[i:i + 1, :]
        scol = scol2[:, i:i + 1]
        beats = (scol > srow) | ((scol == srow) & (qi < pi))
        rank = jnp.sum(jnp.where(beats, 1.0, 0.0), axis=0, keepdims=True)
        sel = jnp.where(rank == rowid.astype(jnp.float32), 1.0, 0.0) * rmask
        sels.append(sel)
        g = jnp.dot(sel, cat_ref[i * _P:(i + 1) * _P, :], precision=_EXACT,
                    preferred_element_type=jnp.float32)
        gath.append(g)

    g2 = jnp.concatenate(gath, axis=0)
    pmu = g2[:, 0:_C]
    plv = g2[:, 128:128 + _C]
    old = g2[:, 256:256 + _C]
    osel = g2[:, 384:384 + _C]

    z = pmu + jnp.exp(0.5 * plv) * eps_ref[...].reshape(_NI * _KPAD, _C)
    h = jnp.maximum(
        jnp.dot(z, r1W_ref[...], preferred_element_type=jnp.float32) + r1b_ref[...],
        0.0)
    rec = jnp.dot(h, r2W_ref[...], preferred_element_type=jnp.float32) + r2b_ref[...]
    newr = 0.5 * rec + 0.5 * old
    diff = newr - old

    for i in range(_NI):
        delta = jax.lax.dot_general(
            sels[i], diff[i * _KPAD:(i + 1) * _KPAD], _TDIMS, precision=_EXACT,
            preferred_element_type=jnp.float32)
        supd_ref[i] = sar2[i] + delta

    rmask2 = jnp.concatenate([rmask] * _NI, axis=0)
    d = newr - osel
    recon_part = jnp.sum(d * d * rmask2)
    kl_part = jnp.sum((1.0 + plv - pmu * pmu - jnp.exp(plv)) * rmask2)

    @pl.when(b == 0)
    def _init():
        acc_ref[0] = 0.0
        acc_ref[1] = 0.0
        acc_ref[2] = 0.0

    acc_ref[0] = acc_ref[0] + recon_part
    acc_ref[1] = acc_ref[1] + kl_part
    acc_ref[2] = acc_ref[2] + align_part

    @pl.when(b == _STEPS - 1)
    def _fin():
        recon_ref[0, 0] = acc_ref[0] * (1.0 / (_B * _KP * _C))
        totkl_ref[0, 0] = (acc_ref[1] * (-0.5 / _B) + acc_ref[2] * (1.0 / _B))


def kernel(opt_token, sar_token, mu_W, mu_b, lv_W, lv_b, fc1_W, fc1_b,
           fc2_W, fc2_b, rec1_W, rec1_b, rec2_W, rec2_b):
    optf = opt_token.reshape(_B, _C, _P).transpose(0, 2, 1)
    sarf = sar_token.reshape(_B, _C, _P).transpose(0, 2, 1)
    eps = jax.random.normal(jax.random.key(42), (_B * _KP, _C), jnp.float32)
    eps_p = jnp.zeros((_B, _KPAD, _C), jnp.float32
                      ).at[:, :_KP].set(eps.reshape(_B, _KP, _C))

    full = lambda *shape: pl.BlockSpec(shape, lambda b: (0,) * len(shape))
    in_specs = [
        pl.BlockSpec((_NI, _P, _C), lambda b: (b, 0, 0)),
        pl.BlockSpec((_NI, _P, _C), lambda b: (b, 0, 0)),
        full(_C, _C), full(1, _C),
        full(_C, _C), full(1, _C),
        full(_P, 128), full(1, 128),
        full(128, _P), full(1, _P),
        full(_C, 128), full(1, 128),
        full(128, _C), full(1, _C),
        pl.BlockSpec((_NI, _KPAD, _C), lambda b: (b, 0, 0)),
    ]
    out_specs = [
        pl.BlockSpec((_NI, _P, _C), lambda b: (b, 0, 0)),
        pl.BlockSpec((_NI, 1, _P), lambda b: (b, 0, 0)),
        pl.BlockSpec((1, 1), lambda b: (0, 0), memory_space=pltpu.SMEM),
        pl.BlockSpec((1, 1), lambda b: (0, 0), memory_space=pltpu.SMEM),
    ]
    out_shapes = [
        jax.ShapeDtypeStruct((_B, _P, _C), jnp.float32),
        jax.ShapeDtypeStruct((_B, 1, _P), jnp.float32),
        jax.ShapeDtypeStruct((1, 1), jnp.float32),
        jax.ShapeDtypeStruct((1, 1), jnp.float32),
    ]
    scratch = [
        pltpu.VMEM((_NI * _P, 512), jnp.float32),
        pltpu.SMEM((3,), jnp.float32),
    ]
    sar_upd, scores, recon, totkl = pl.pallas_call(
        _body,
        grid=(_STEPS,),
        in_specs=in_specs,
        out_specs=out_specs,
        out_shape=out_shapes,
        scratch_shapes=scratch,
        compiler_params=pltpu.CompilerParams(
            dimension_semantics=("arbitrary",)),
    )(optf, sarf, mu_W, mu_b.reshape(1, _C), lv_W, lv_b.reshape(1, _C),
      fc1_W, fc1_b.reshape(1, 128), fc2_W, fc2_b.reshape(1, _P),
      rec1_W, rec1_b.reshape(1, 128), rec2_W, rec2_b.reshape(1, _C),
      eps_p)

    unc_map = scores.reshape(_B, 1, _H, _W)
    return (optf, sar_upd, recon[0, 0], totkl[0, 0], unc_map)


# speedup vs baseline: 3.6725x; 1.0725x over previous
import jax
import jax.numpy as jnp
import numpy as np
from jax.experimental import pallas as pl
from jax.experimental.pallas import tpu as pltpu

_B, _C, _H, _W = 16, 96, 24, 24
_P = _H * _W
_KP = 500 // _B
_KPAD = 32
_NI = 4
_STEPS = _B // _NI

_EXACT = jax.lax.Precision.HIGHEST
_TDIMS = (((0,), (0,)), ((), ()))


def _body(optf_ref, sarf_ref, muW_ref, mub_ref, lvW_ref, lvb_ref,
          f1W_ref, f1b_ref, f2W_ref, f2b_ref, r1W_ref, r1b_ref,
          r2W_ref, r2b_ref, eps_ref,
          supd_ref, sc_ref, recon_ref, totkl_ref, cat_ref, acc_ref):
    b = pl.program_id(0)
    opt2 = optf_ref[...]
    sar2 = sarf_ref[...]
    opt = opt2.reshape(_NI * _P, _C)
    sar = sar2.reshape(_NI * _P, _C)
    muW = muW_ref[...]
    mub = mub_ref[...]
    lvW = lvW_ref[...]
    lvb = lvb_ref[...]

    omu = jnp.dot(opt, muW, preferred_element_type=jnp.float32) + mub
    olv = jnp.dot(opt, lvW, preferred_element_type=jnp.float32) + lvb
    smu = jnp.dot(sar, muW, preferred_element_type=jnp.float32) + mub
    slv = jnp.clip(jnp.dot(sar, lvW, preferred_element_type=jnp.float32) + lvb,
                   -10.0, 10.0)

    sum_o = jnp.sum(olv, axis=1, keepdims=True)
    sum_s = jnp.sum(slv, axis=1, keepdims=True)
    vcol = 0.5 * jnp.log((jnp.exp(sum_s) + 1e-6) / (jnp.exp(sum_o) + 1e-6))

    h1s = [
        jnp.maximum(
            jax.lax.dot_general(vcol[i * _P:(i + 1) * _P], f1W_ref[...],
                                _TDIMS, preferred_element_type=jnp.float32)
            + f1b_ref[...], 0.0)
        for i in range(_NI)
    ]
    raw = (jnp.dot(jnp.concatenate(h1s, axis=0), f2W_ref[...],
                   preferred_element_type=jnp.float32)
           + f2b_ref[...])
    mx = jnp.max(raw, axis=1, keepdims=True)
    ex = jnp.exp(raw - mx)
    scores = ex / jnp.sum(ex, axis=1, keepdims=True)
    sc_ref[...] = scores.reshape(_NI, 1, _P)

    pmx = jnp.max(omu, axis=1, keepdims=True)
    pex = jnp.exp(omu - pmx)
    p = pex / jnp.sum(pex, axis=1, keepdims=True)
    qmx = jnp.max(smu, axis=1, keepdims=True)
    logq = (smu - qmx) - jnp.log(jnp.sum(jnp.exp(smu - qmx), axis=1, keepdims=True))
    align_part = jnp.sum(p * (jnp.log(p) - logq))

    scol2 = jnp.transpose(scores)

    cat_ref[:, 0:_C] = omu
    cat_ref[:, 128:128 + _C] = olv
    cat_ref[:, 256:256 + _C] = sar
    cat_ref[:, 384:384 + _C] = opt

    qi = jax.lax.broadcasted_iota(jnp.int32, (_P, 1), 0)
    pi = jax.lax.broadcasted_iota(jnp.int32, (1, _P), 1)
    rowid = jax.lax.broadcasted_iota(jnp.int32, (_KPAD, 1), 0)
    rmask = (rowid < _KP).astype(jnp.float32)

    sels = []
    gath = []
    for i in range(_NI):
        srow = scores[i:i + 1, :]
        scol = scol2[:, i:i + 1]
        beats = (scol > srow) | ((scol == srow) & (qi < pi))
        rank = jnp.sum(jnp.where(beats, 1.0, 0.0), axis=0, keepdims=True)
        sel = jnp.where(rank == rowid.astype(jnp.float32), 1.0, 0.0) * rmask
        sels.append(sel)
        g = jnp.dot(sel, cat_ref[i * _P:(i + 1) * _P, :], precision=_EXACT,
                    preferred_element_type=jnp.float32)
        gath.append(g)

    g2 = jnp.concatenate(gath, axis=0)
    pmu = g2[:, 0:_C]
    plv = g2[:, 128:128 + _C]
    old = g2[:, 256:256 + _C]
    osel = g2[:, 384:384 + _C]

    z = pmu + jnp.exp(0.5 * plv) * eps_ref[...].reshape(_NI * _KPAD, _C)
    h = jnp.maximum(
        jnp.dot(z, r1W_ref[...], preferred_element_type=jnp.float32) + r1b_ref[...],
        0.0)
    rec = jnp.dot(h, r2W_ref[...], preferred_element_type=jnp.float32) + r2b_ref[...]
    newr = 0.5 * rec + 0.5 * old
    diff = newr - old

    for i in range(_NI):
        delta = jax.lax.dot_general(
            sels[i], diff[i * _KPAD:(i + 1) * _KPAD], _TDIMS, precision=_EXACT,
            preferred_element_type=jnp.float32)
        supd_ref[i] = sar2[i] + delta

    rmask2 = jnp.concatenate([rmask] * _NI, axis=0)
    d = newr - osel
    recon_part = jnp.sum(d * d * rmask2)
    kl_part = jnp.sum((1.0 + plv - pmu * pmu - jnp.exp(plv)) * rmask2)

    @pl.when(b == 0)
    def _init():
        acc_ref[0] = 0.0
        acc_ref[1] = 0.0
        acc_ref[2] = 0.0

    acc_ref[0] = acc_ref[0] + recon_part
    acc_ref[1] = acc_ref[1] + kl_part
    acc_ref[2] = acc_ref[2] + align_part

    @pl.when(b == _STEPS - 1)
    def _fin():
        recon_ref[0, 0] = acc_ref[0] * (1.0 / (_B * _KP * _C))
        totkl_ref[0, 0] = (acc_ref[1] * (-0.5 / _B) + acc_ref[2] * (1.0 / _B))


def kernel(opt_token, sar_token, mu_W, mu_b, lv_W, lv_b, fc1_W, fc1_b,
           fc2_W, fc2_b, rec1_W, rec1_b, rec2_W, rec2_b):
    optf = opt_token.reshape(_B, _C, _P).transpose(0, 2, 1)
    sarf = sar_token.reshape(_B, _C, _P).transpose(0, 2, 1)
    eps = jax.random.normal(jax.random.key(42), (_B * _KP, _C), jnp.float32)
    eps_p = jnp.zeros((_B, _KPAD, _C), jnp.float32
                      ).at[:, :_KP].set(eps.reshape(_B, _KP, _C))

    full = lambda *shape: pl.BlockSpec(shape, lambda b: (0,) * len(shape))
    in_specs = [
        pl.BlockSpec((_NI, _P, _C), lambda b: (b, 0, 0)),
        pl.BlockSpec((_NI, _P, _C), lambda b: (b, 0, 0)),
        full(_C, _C), full(1, _C),
        full(_C, _C), full(1, _C),
        full(_P, 128), full(1, 128),
        full(128, _P), full(1, _P),
        full(_C, 128), full(1, 128),
        full(128, _C), full(1, _C),
        pl.BlockSpec((_NI, _KPAD, _C), lambda b: (b, 0, 0)),
    ]
    out_specs = [
        pl.BlockSpec((_NI, _P, _C), lambda b: (b, 0, 0)),
        pl.BlockSpec((_NI, 1, _P), lambda b: (b, 0, 0)),
        pl.BlockSpec((1, 1), lambda b: (0, 0), memory_space=pltpu.SMEM),
        pl.BlockSpec((1, 1), lambda b: (0, 0), memory_space=pltpu.SMEM),
    ]
    out_shapes = [
        jax.ShapeDtypeStruct((_B, _P, _C), jnp.float32),
        jax.ShapeDtypeStruct((_B, 1, _P), jnp.float32),
        jax.ShapeDtypeStruct((1, 1), jnp.float32),
        jax.ShapeDtypeStruct((1, 1), jnp.float32),
    ]
    scratch = [
        pltpu.VMEM((_NI * _P, 512), jnp.float32),
        pltpu.SMEM((3,), jnp.float32),
    ]
    sar_upd, scores, recon, totkl = pl.pallas_call(
        _body,
        grid=(_STEPS,),
        in_specs=in_specs,
        out_specs=out_specs,
        out_shape=out_shapes,
        scratch_shapes=scratch,
        compiler_params=pltpu.CompilerParams(
            dimension_semantics=("arbitrary",)),
    )(optf, sarf, mu_W, mu_b.reshape(1, _C), lv_W, lv_b.reshape(1, _C),
      fc1_W, fc1_b.reshape(1, 128), fc2_W, fc2_b.reshape(1, _P),
      rec1_W, rec1_b.reshape(1, 128), rec2_W, rec2_b.reshape(1, _C),
      eps_p)

    unc_map = scores.reshape(_B, 1, _H, _W)
    return (optf, sar_upd, recon[0, 0], totkl[0, 0], unc_map)


# speedup vs baseline: 3.7249x; 1.0143x over previous
import jax
import jax.numpy as jnp
import numpy as np
from jax.experimental import pallas as pl
from jax.experimental.pallas import tpu as pltpu

_B, _C, _H, _W = 16, 96, 24, 24
_P = _H * _W
_KP = 500 // _B
_KPAD = 32
_NI = 8
_STEPS = _B // _NI

_EXACT = jax.lax.Precision.HIGHEST
_TDIMS = (((0,), (0,)), ((), ()))


def _body(optf_ref, sarf_ref, muW_ref, mub_ref, lvW_ref, lvb_ref,
          f1W_ref, f1b_ref, f2W_ref, f2b_ref, r1W_ref, r1b_ref,
          r2W_ref, r2b_ref, eps_ref,
          supd_ref, sc_ref, recon_ref, totkl_ref, cat_ref, acc_ref):
    b = pl.program_id(0)
    opt2 = optf_ref[...]
    sar2 = sarf_ref[...]
    opt = opt2.reshape(_NI * _P, _C)
    sar = sar2.reshape(_NI * _P, _C)
    muW = muW_ref[...]
    mub = mub_ref[...]
    lvW = lvW_ref[...]
    lvb = lvb_ref[...]

    omu = jnp.dot(opt, muW, preferred_element_type=jnp.float32) + mub
    olv = jnp.dot(opt, lvW, preferred_element_type=jnp.float32) + lvb
    smu = jnp.dot(sar, muW, preferred_element_type=jnp.float32) + mub
    slv = jnp.clip(jnp.dot(sar, lvW, preferred_element_type=jnp.float32) + lvb,
                   -10.0, 10.0)

    sum_o = jnp.sum(olv, axis=1, keepdims=True)
    sum_s = jnp.sum(slv, axis=1, keepdims=True)
    vcol = 0.5 * jnp.log((jnp.exp(sum_s) + 1e-6) / (jnp.exp(sum_o) + 1e-6))

    h1s = [
        jnp.maximum(
            jax.lax.dot_general(vcol[i * _P:(i + 1) * _P], f1W_ref[...],
                                _TDIMS, preferred_element_type=jnp.float32)
            + f1b_ref[...], 0.0)
        for i in range(_NI)
    ]
    raw = (jnp.dot(jnp.concatenate(h1s, axis=0), f2W_ref[...],
                   preferred_element_type=jnp.float32)
           + f2b_ref[...])
    mx = jnp.max(raw, axis=1, keepdims=True)
    ex = jnp.exp(raw - mx)
    scores = ex / jnp.sum(ex, axis=1, keepdims=True)
    sc_ref[...] = scores.reshape(_NI, 1, _P)

    pmx = jnp.max(omu, axis=1, keepdims=True)
    pex = jnp.exp(omu - pmx)
    p = pex / jnp.sum(pex, axis=1, keepdims=True)
    qmx = jnp.max(smu, axis=1, keepdims=True)
    logq = (smu - qmx) - jnp.log(jnp.sum(jnp.exp(smu - qmx), axis=1, keepdims=True))
    align_part = jnp.sum(p * (jnp.log(p) - logq))

    scol2 = jnp.transpose(scores)

    cat_ref[:, 0:_C] = omu
    cat_ref[:, 128:128 + _C] = olv
    cat_ref[:, 256:256 + _C] = sar
    cat_ref[:, 384:384 + _C] = opt

    qi = jax.lax.broadcasted_iota(jnp.int32, (_P, 1), 0)
    pi = jax.lax.broadcasted_iota(jnp.int32, (1, _P), 1)
    rowid = jax.lax.broadcasted_iota(jnp.int32, (_KPAD, 1), 0)
    rmask = (rowid < _KP).astype(jnp.float32)

    sels = []
    gath = []
    for i in range(_NI):
        srow = scores[i:i + 1, :]
        scol = scol2[:, i:i + 1]
        beats = (scol > srow) | ((scol == srow) & (qi < pi))
        rank = jnp.sum(jnp.where(beats, 1.0, 0.0), axis=0, keepdims=True)
        sel = jnp.where(rank == rowid.astype(jnp.float32), 1.0, 0.0) * rmask
        sels.append(sel)
        g = jnp.dot(sel, cat_ref[i * _P:(i + 1) * _P, :], precision=_EXACT,
                    preferred_element_type=jnp.float32)
        gath.append(g)

    g2 = jnp.concatenate(gath, axis=0)
    pmu = g2[:, 0:_C]
    plv = g2[:, 128:128 + _C]
    old = g2[:, 256:256 + _C]
    osel = g2[:, 384:384 + _C]

    z = pmu + jnp.exp(0.5 * plv) * eps_ref[...].reshape(_NI * _KPAD, _C)
    h = jnp.maximum(
        jnp.dot(z, r1W_ref[...], preferred_element_type=jnp.float32) + r1b_ref[...],
        0.0)
    rec = jnp.dot(h, r2W_ref[...], preferred_element_type=jnp.float32) + r2b_ref[...]
    newr = 0.5 * rec + 0.5 * old
    diff = newr - old

    for i in range(_NI):
        delta = jax.lax.dot_general(
            sels[i], diff[i * _KPAD:(i + 1) * _KPAD], _TDIMS, precision=_EXACT,
            preferred_element_type=jnp.float32)
        supd_ref[i] = sar2[i] + delta

    rmask2 = jnp.concatenate([rmask] * _NI, axis=0)
    d = newr - osel
    recon_part = jnp.sum(d * d * rmask2)
    kl_part = jnp.sum((1.0 + plv - pmu * pmu - jnp.exp(plv)) * rmask2)

    @pl.when(b == 0)
    def _init():
        acc_ref[0] = 0.0
        acc_ref[1] = 0.0
        acc_ref[2] = 0.0

    acc_ref[0] = acc_ref[0] + recon_part
    acc_ref[1] = acc_ref[1] + kl_part
    acc_ref[2] = acc_ref[2] + align_part

    @pl.when(b == _STEPS - 1)
    def _fin():
        recon_ref[0, 0] = acc_ref[0] * (1.0 / (_B * _KP * _C))
        totkl_ref[0, 0] = (acc_ref[1] * (-0.5 / _B) + acc_ref[2] * (1.0 / _B))


def kernel(opt_token, sar_token, mu_W, mu_b, lv_W, lv_b, fc1_W, fc1_b,
           fc2_W, fc2_b, rec1_W, rec1_b, rec2_W, rec2_b):
    optf = opt_token.reshape(_B, _C, _P).transpose(0, 2, 1)
    sarf = sar_token.reshape(_B, _C, _P).transpose(0, 2, 1)
    eps = jax.random.normal(jax.random.key(42), (_B * _KP, _C), jnp.float32)
    eps_p = jnp.zeros((_B, _KPAD, _C), jnp.float32
                      ).at[:, :_KP].set(eps.reshape(_B, _KP, _C))

    full = lambda *shape: pl.BlockSpec(shape, lambda b: (0,) * len(shape))
    in_specs = [
        pl.BlockSpec((_NI, _P, _C), lambda b: (b, 0, 0)),
        pl.BlockSpec((_NI, _P, _C), lambda b: (b, 0, 0)),
        full(_C, _C), full(1, _C),
        full(_C, _C), full(1, _C),
        full(_P, 128), full(1, 128),
        full(128, _P), full(1, _P),
        full(_C, 128), full(1, 128),
        full(128, _C), full(1, _C),
        pl.BlockSpec((_NI, _KPAD, _C), lambda b: (b, 0, 0)),
    ]
    out_specs = [
        pl.BlockSpec((_NI, _P, _C), lambda b: (b, 0, 0)),
        pl.BlockSpec((_NI, 1, _P), lambda b: (b, 0, 0)),
        pl.BlockSpec((1, 1), lambda b: (0, 0), memory_space=pltpu.SMEM),
        pl.BlockSpec((1, 1), lambda b: (0, 0), memory_space=pltpu.SMEM),
    ]
    out_shapes = [
        jax.ShapeDtypeStruct((_B, _P, _C), jnp.float32),
        jax.ShapeDtypeStruct((_B, 1, _P), jnp.float32),
        jax.ShapeDtypeStruct((1, 1), jnp.float32),
        jax.ShapeDtypeStruct((1, 1), jnp.float32),
    ]
    scratch = [
        pltpu.VMEM((_NI * _P, 512), jnp.float32),
        pltpu.SMEM((3,), jnp.float32),
    ]
    sar_upd, scores, recon, totkl = pl.pallas_call(
        _body,
        grid=(_STEPS,),
        in_specs=in_specs,
        out_specs=out_specs,
        out_shape=out_shapes,
        scratch_shapes=scratch,
        compiler_params=pltpu.CompilerParams(
            dimension_semantics=("arbitrary",)),
    )(optf, sarf, mu_W, mu_b.reshape(1, _C), lv_W, lv_b.reshape(1, _C),
      fc1_W, fc1_b.reshape(1, 128), fc2_W, fc2_b.reshape(1, _P),
      rec1_W, rec1_b.reshape(1, 128), rec2_W, rec2_b.reshape(1, _C),
      eps_p)

    unc_map = scores.reshape(_B, 1, _H, _W)
    return (optf, sar_upd, recon[0, 0], totkl[0, 0], unc_map)
